# Initial kernel scaffold; baseline (speedup 1.0000x reference)
import jax
import jax.numpy as jnp
from jax.experimental import pallas as pl


def kernel(z, embeddings, W1, b1, W2, b2, W3, b3, W4, b4):
    raise NotImplementedError("write your pallas kernel here")


# speedup vs baseline: 2.6074x; 2.6074x over previous
import jax
import jax.numpy as jnp
from jax.experimental import pallas as pl
from jax.experimental.pallas import tpu as pltpu

VOCAB = 1024
E = 256
K = 8
T = 4
H = 4 * E


def _layernorm(x, eps=1e-5):
    m = jnp.mean(x, axis=-1, keepdims=True)
    v = jnp.mean((x - m) ** 2, axis=-1, keepdims=True)
    return (x - m) / jnp.sqrt(v + eps)


def _body(emb_ref, q_ref, w1_ref, b1_ref, w2_ref, b2_ref, w3_ref, b3_ref,
          w4_ref, b4_ref, tok_ref, out_ref):
    x = emb_ref[0]
    h = jnp.maximum(jnp.dot(x, w1_ref[...]) + b1_ref[...], 0.0)
    h = jnp.maximum(jnp.dot(h, w2_ref[...]) + b2_ref[...], 0.0)
    h = jnp.maximum(jnp.dot(h, w3_ref[...]) + b3_ref[...], 0.0)
    mem = jnp.dot(h, w4_ref[...]) + b4_ref[...]
    mem = _layernorm(mem)

    q = _layernorm(q_ref[0]) * (E ** -0.5)
    s = jax.lax.dot_general(q, mem, (((1,), (1,)), ((), ())))
    p = jnp.exp(s - jnp.max(s, axis=-1, keepdims=True))
    p = p / jnp.sum(p, axis=-1, keepdims=True)
    tok_ref[0, 0, :] = jnp.argmax(p, axis=-1).astype(jnp.int32)
    out_ref[0] = jnp.dot(p, mem)


def kernel(z, embeddings, W1, b1, W2, b2, W3, b3, W4, b4):
    b = z.shape[0] // (K * T)
    bk = b * K
    emb_t = embeddings.reshape(VOCAB, T, E).transpose(1, 0, 2)
    z_t = z.reshape(b, K, T, E).transpose(2, 0, 1, 3).reshape(T, bk, E)
    b1r, b2r, b3r = b1.reshape(1, H), b2.reshape(1, H), b3.reshape(1, H)
    b4r = b4.reshape(1, E)

    tok_t, out_t = pl.pallas_call(
        _body,
        grid=(T,),
        in_specs=[
            pl.BlockSpec((1, VOCAB, E), lambda t: (t, 0, 0)),
            pl.BlockSpec((1, bk, E), lambda t: (t, 0, 0)),
            pl.BlockSpec((E, H), lambda t: (0, 0)),
            pl.BlockSpec((1, H), lambda t: (0, 0)),
            pl.BlockSpec((H, H), lambda t: (0, 0)),
            pl.BlockSpec((1, H), lambda t: (0, 0)),
            pl.BlockSpec((H, H), lambda t: (0, 0)),
            pl.BlockSpec((1, H), lambda t: (0, 0)),
            pl.BlockSpec((H, E), lambda t: (0, 0)),
            pl.BlockSpec((1, E), lambda t: (0, 0)),
        ],
        out_specs=[
            pl.BlockSpec((1, 1, bk), lambda t: (t, 0, 0)),
            pl.BlockSpec((1, bk, E), lambda t: (t, 0, 0)),
        ],
        out_shape=[
            jax.ShapeDtypeStruct((T, 1, bk), jnp.int32),
            jax.ShapeDtypeStruct((T, bk, E), jnp.float32),
        ],
    )(emb_t, z_t, W1, b1r, W2, b2r, W3, b3r, W4, b4r)

    tokens = tok_t.reshape(T, bk).T.reshape(bk * T)
    z_q = out_t.reshape(T, b, K, E).transpose(1, 2, 0, 3).reshape(bk * T, E)
    return (tokens, z_q)


# speedup vs baseline: 2.7509x; 1.0551x over previous
import jax
import jax.numpy as jnp
from jax.experimental import pallas as pl
from jax.experimental.pallas import tpu as pltpu

VOCAB = 1024
E = 256
K = 8
T = 4
H = 4 * E


def _layernorm(x, eps=1e-5):
    m = jnp.mean(x, axis=-1, keepdims=True)
    v = jnp.mean((x - m) ** 2, axis=-1, keepdims=True)
    return (x - m) / jnp.sqrt(v + eps)


def _body(emb_ref, q_ref, w1_ref, b1_ref, w2_ref, b2_ref, w3_ref, b3_ref,
          w4_ref, b4_ref, tok_ref, out_ref):
    x = emb_ref[...]
    h = jnp.maximum(jnp.dot(x, w1_ref[...]) + b1_ref[...], 0.0)
    h = jnp.maximum(jnp.dot(h, w2_ref[...]) + b2_ref[...], 0.0)
    h = jnp.maximum(jnp.dot(h, w3_ref[...]) + b3_ref[...], 0.0)
    mem = jnp.dot(h, w4_ref[...]) + b4_ref[...]
    mem = _layernorm(mem)

    q = _layernorm(q_ref[...]) * (E ** -0.5)
    s = jax.lax.dot_general(q, mem, (((1,), (1,)), ((), ())))
    p = jnp.exp(s - jnp.max(s, axis=-1, keepdims=True))
    p = p / jnp.sum(p, axis=-1, keepdims=True)
    tok_ref[0, 0, :] = jnp.argmax(p, axis=-1).astype(jnp.int32)
    out_ref[...] = jnp.dot(p, mem)


def kernel(z, embeddings, W1, b1, W2, b2, W3, b3, W4, b4):
    b = z.shape[0] // (K * T)
    bk = b * K
    emb_v = embeddings.reshape(VOCAB, T * E)
    z_v = z.reshape(bk, T * E)
    b1r, b2r, b3r = b1.reshape(1, H), b2.reshape(1, H), b3.reshape(1, H)
    b4r = b4.reshape(1, E)

    tok_t, out_v = pl.pallas_call(
        _body,
        grid=(T,),
        in_specs=[
            pl.BlockSpec((VOCAB, E), lambda t: (0, t)),
            pl.BlockSpec((bk, E), lambda t: (0, t)),
            pl.BlockSpec((E, H), lambda t: (0, 0)),
            pl.BlockSpec((1, H), lambda t: (0, 0)),
            pl.BlockSpec((H, H), lambda t: (0, 0)),
            pl.BlockSpec((1, H), lambda t: (0, 0)),
            pl.BlockSpec((H, H), lambda t: (0, 0)),
            pl.BlockSpec((1, H), lambda t: (0, 0)),
            pl.BlockSpec((H, E), lambda t: (0, 0)),
            pl.BlockSpec((1, E), lambda t: (0, 0)),
        ],
        out_specs=[
            pl.BlockSpec((1, 1, bk), lambda t: (t, 0, 0)),
            pl.BlockSpec((bk, E), lambda t: (0, t)),
        ],
        out_shape=[
            jax.ShapeDtypeStruct((T, 1, bk), jnp.int32),
            jax.ShapeDtypeStruct((bk, T * E), jnp.float32),
        ],
    )(emb_v, z_v, W1, b1r, W2, b2r, W3, b3r, W4, b4r)

    tokens = tok_t.reshape(T, bk).T.reshape(bk * T)
    z_q = out_v.reshape(bk * T, E)
    return (tokens, z_q)


# speedup vs baseline: 3.0145x; 1.0958x over previous
import jax
import jax.numpy as jnp
from jax.experimental import pallas as pl
from jax.experimental.pallas import tpu as pltpu

VOCAB = 1024
E = 256
K = 8
T = 4
H = 4 * E
VB = 256
QB = 512


def _layernorm(x, eps=1e-5):
    m = jnp.mean(x, axis=-1, keepdims=True)
    v = jnp.mean((x - m) ** 2, axis=-1, keepdims=True)
    return (x - m) / jnp.sqrt(v + eps)


def _body(emb_ref, z_ref, w1_ref, b1_ref, w2_ref, b2_ref, w3_ref, b3_ref,
          w4_ref, b4_ref, tok_ref, out_ref, mem_ref):
    i = pl.program_id(0)

    @pl.when(i < T)
    def _mlp():
        x = emb_ref[...].reshape(VB * T, E)
        h = jnp.maximum(jnp.dot(x, w1_ref[...]) + b1_ref[...], 0.0)
        h = jnp.maximum(jnp.dot(h, w2_ref[...]) + b2_ref[...], 0.0)
        h = jnp.maximum(jnp.dot(h, w3_ref[...]) + b3_ref[...], 0.0)
        mem = jnp.dot(h, w4_ref[...]) + b4_ref[...]
        mem = _layernorm(mem).reshape(VB, T, E)
        for t in range(T):
            mem_ref[t, pl.ds(i * VB, VB), :] = mem[:, t, :]

    @pl.when(i >= T)
    def _attn():
        q = _layernorm(z_ref[...]) * (E ** -0.5)
        qr = q.reshape(QB // T, T, E)
        outs = []
        for t in range(T):
            qt = qr[:, t, :]
            mt = mem_ref[t]
            s = jax.lax.dot_general(qt, mt, (((1,), (1,)), ((), ())))
            p = jnp.exp(s - jnp.max(s, axis=-1, keepdims=True))
            p = p / jnp.sum(p, axis=-1, keepdims=True)
            tok_ref[0, t, :] = jnp.argmax(p, axis=-1).astype(jnp.int32)
            outs.append(jnp.dot(p, mt))
        out_ref[...] = jnp.stack(outs, axis=1).reshape(QB, E)


def kernel(z, embeddings, W1, b1, W2, b2, W3, b3, W4, b4):
    n = z.shape[0]
    emb3 = embeddings.reshape(VOCAB, T, E)
    b1r, b2r, b3r = b1.reshape(1, H), b2.reshape(1, H), b3.reshape(1, H)
    b4r = b4.reshape(1, E)
    nq = n // QB

    tok_t, z_q = pl.pallas_call(
        _body,
        grid=(T + nq,),
        in_specs=[
            pl.BlockSpec((VB, T, E), lambda i: (jnp.minimum(i, T - 1), 0, 0)),
            pl.BlockSpec((QB, E), lambda i: (jnp.maximum(i - T, 0), 0)),
            pl.BlockSpec((E, H), lambda i: (0, 0)),
            pl.BlockSpec((1, H), lambda i: (0, 0)),
            pl.BlockSpec((H, H), lambda i: (0, 0)),
            pl.BlockSpec((1, H), lambda i: (0, 0)),
            pl.BlockSpec((H, H), lambda i: (0, 0)),
            pl.BlockSpec((1, H), lambda i: (0, 0)),
            pl.BlockSpec((H, E), lambda i: (0, 0)),
            pl.BlockSpec((1, E), lambda i: (0, 0)),
        ],
        out_specs=[
            pl.BlockSpec((1, T, QB // T), lambda i: (jnp.maximum(i - T, 0), 0, 0)),
            pl.BlockSpec((QB, E), lambda i: (jnp.maximum(i - T, 0), 0)),
        ],
        out_shape=[
            jax.ShapeDtypeStruct((nq, T, QB // T), jnp.int32),
            jax.ShapeDtypeStruct((n, E), jnp.float32),
        ],
        scratch_shapes=[pltpu.VMEM((T, VOCAB, E), jnp.float32)],
    )(emb3, z, W1, b1r, W2, b2r, W3, b3r, W4, b4r)

    tokens = tok_t.transpose(0, 2, 1).reshape(n)
    return (tokens, z_q)
